# baseline (speedup 1.0000x reference)
import functools
import math

import jax
import jax.numpy as jnp
from jax import lax
from jax.experimental import pallas as pl
from jax.experimental.pallas import tpu as pltpu
from jax.experimental.pallas import tpu_sc as plsc

D_MODEL = 768
SCALE = math.sqrt(float(D_MODEL))
LANES = 16


@functools.cache
def _make_kernel(SEQ, BATCH, V, D):
    B = SEQ * BATCH
    info = plsc.get_sparse_core_info()
    NC, NS = info.num_cores, info.num_subcores
    NW = NC * NS
    b_per_w = B // NW
    CH = 32
    n_ch = b_per_w // CH
    CH_S = CH // BATCH
    NVEC = D // LANES

    mesh = plsc.VectorSubcoreMesh(core_axis_name="c", subcore_axis_name="s")

    @functools.partial(
        pl.kernel,
        mesh=mesh,
        out_type=jax.ShapeDtypeStruct((SEQ, BATCH, D), jnp.float32),
        scratch_types=[
            pltpu.VMEM((2, CH), jnp.int32),
            pltpu.VMEM((CH, D), jnp.float32),
            pltpu.VMEM((CH, D), jnp.float32),
            pltpu.VMEM((CH_S, BATCH, D), jnp.float32),
            pltpu.VMEM((CH_S, BATCH, D), jnp.float32),
            pltpu.VMEM((CH_S * D,), jnp.float32),
            pltpu.VMEM((CH_S * D,), jnp.float32),
            pltpu.SemaphoreType.DMA,
            pltpu.SemaphoreType.DMA,
            pltpu.SemaphoreType.DMA,
            pltpu.SemaphoreType.DMA,
            pltpu.SemaphoreType.DMA,
            pltpu.SemaphoreType.DMA,
        ],
    )
    def k(x_hbm, table_hbm, pe_hbm, out_hbm,
          idx_v, rows0, rows1, ov0, ov1, pe0, pe1,
          sg0, sg1, sp0, sp1, so0, so1):
        rows = [rows0, rows1]
        outv = [ov0, ov1]
        pev = [pe0, pe1]
        sg = [sg0, sg1]
        sp = [sp0, sp1]
        so = [so0, so1]
        wid = lax.axis_index("s") * NC + lax.axis_index("c")
        base = wid * b_per_w

        gathers = [None, None]
        pes = [None, None]
        outs = [None, None]

        def start(c):
            b = c & 1
            cbase = pl.multiple_of(base + c * CH, CH)
            pltpu.sync_copy(x_hbm.at[pl.ds(cbase, CH)], idx_v.at[b])
            gathers[b] = pltpu.async_copy(table_hbm.at[idx_v.at[b]], rows[b], sg[b])
            pes[b] = pltpu.async_copy(
                pe_hbm.at[pl.ds(pl.multiple_of((cbase // BATCH) * D, CH_S * D),
                                CH_S * D)],
                pev[b], sp[b],
            )

        def compute(rows_v, pe_v, out_v):
            def s_body(si, carry2):
                def d_body(di, carry3):
                    sl = pl.ds(di * LANES, LANES)
                    pvec = pe_v[pl.ds(si * D + di * LANES, LANES)]
                    for b in range(BATCH):
                        out_v[si, b, sl] = rows_v[si * BATCH + b, sl] * SCALE + pvec
                    return carry3

                return lax.fori_loop(0, NVEC, d_body, carry2)

            lax.fori_loop(0, CH_S, s_body, 0)

        start(0)
        for c in range(n_ch):
            b = c & 1
            nb = 1 - b
            if c + 1 < n_ch:
                start(c + 1)
            gathers[b].wait()
            pes[b].wait()
            if c >= 2:
                outs[b].wait()
            compute(rows[b], pev[b], outv[b])
            sbase = pl.multiple_of((base + c * CH) // BATCH, CH_S)
            outs[b] = pltpu.async_copy(outv[b], out_hbm.at[pl.ds(sbase, CH_S)], so[b])
        outs[(n_ch - 2) & 1].wait()
        outs[(n_ch - 1) & 1].wait()

    return k


def kernel(x, table, pe):
    seq, batch = x.shape
    xf = x.reshape(seq * batch)
    pef = pe.reshape(pe.shape[0] * pe.shape[2])
    out = _make_kernel(seq, batch, table.shape[0], table.shape[1])(xf, table, pef)
    return out


# speedup vs baseline: 1.2882x; 1.2882x over previous
import functools
import math

import jax
import jax.numpy as jnp
from jax import lax
from jax.experimental import pallas as pl
from jax.experimental.pallas import tpu as pltpu
from jax.experimental.pallas import tpu_sc as plsc

D_MODEL = 768
SCALE = math.sqrt(float(D_MODEL))
LANES = 16


@functools.cache
def _make_kernel(SEQ, BATCH, V, D):
    B = SEQ * BATCH
    info = plsc.get_sparse_core_info()
    NC, NS = info.num_cores, info.num_subcores
    NW = NC * NS
    b_per_w = B // NW
    CH = 64
    n_ch = b_per_w // CH
    CH_S = CH // BATCH
    NVEC = D // LANES

    mesh = plsc.VectorSubcoreMesh(core_axis_name="c", subcore_axis_name="s")

    @functools.partial(
        pl.kernel,
        mesh=mesh,
        out_type=jax.ShapeDtypeStruct((B, D), jnp.float32),
        scratch_types=[
            pltpu.VMEM((2, CH), jnp.int32),
            pltpu.VMEM((CH, D), jnp.float32),
            pltpu.VMEM((CH, D), jnp.float32),
            pltpu.VMEM((CH_S * D,), jnp.float32),
            pltpu.VMEM((CH_S * D,), jnp.float32),
            pltpu.SemaphoreType.DMA,
            pltpu.SemaphoreType.DMA,
            pltpu.SemaphoreType.DMA,
            pltpu.SemaphoreType.DMA,
            pltpu.SemaphoreType.DMA,
            pltpu.SemaphoreType.DMA,
        ],
    )
    def k(x_hbm, table_hbm, pe_hbm, out_hbm,
          idx_v, rows0, rows1, pe0, pe1, sg0, sg1, sp0, sp1, so0, so1):
        rows = [rows0, rows1]
        pev = [pe0, pe1]
        sg = [sg0, sg1]
        sp = [sp0, sp1]
        so = [so0, so1]
        wid = lax.axis_index("s") * NC + lax.axis_index("c")
        base = wid * b_per_w

        gathers = [None, None]
        pes = [None, None]
        outs = [None, None]

        def start(c):
            b = c & 1
            cbase = pl.multiple_of(base + c * CH, CH)
            pltpu.sync_copy(x_hbm.at[pl.ds(cbase, CH)], idx_v.at[b])
            gathers[b] = pltpu.async_copy(table_hbm.at[idx_v.at[b]], rows[b], sg[b])
            pes[b] = pltpu.async_copy(
                pe_hbm.at[pl.ds(pl.multiple_of((cbase // BATCH) * D, CH_S * D),
                                CH_S * D)],
                pev[b], sp[b],
            )

        def compute(rows_v, pe_v):
            def s_body(si, carry2):
                def d_body(di, carry3):
                    sl = pl.ds(di * LANES, LANES)
                    pvec = pe_v[pl.ds(si * D + di * LANES, LANES)]
                    for b in range(BATCH):
                        r = si * BATCH + b
                        rows_v[r, sl] = rows_v[r, sl] * SCALE + pvec
                    return carry3

                return lax.fori_loop(0, NVEC, d_body, carry2)

            lax.fori_loop(0, CH_S, s_body, 0)

        start(0)
        for c in range(n_ch):
            b = c & 1
            nb = 1 - b
            if c + 1 < n_ch:
                if c >= 1:
                    outs[nb].wait()
                start(c + 1)
            gathers[b].wait()
            pes[b].wait()
            compute(rows[b], pev[b])
            cbase = pl.multiple_of(base + c * CH, CH)
            outs[b] = pltpu.async_copy(rows[b], out_hbm.at[pl.ds(cbase, CH)], so[b])
        outs[(n_ch - 2) & 1].wait()
        outs[(n_ch - 1) & 1].wait()

    return k


def kernel(x, table, pe):
    seq, batch = x.shape
    B = seq * batch
    xf = x.reshape(B)
    pef = pe.reshape(pe.shape[0] * pe.shape[2])
    out = _make_kernel(seq, batch, table.shape[0], table.shape[1])(xf, table, pef)
    return out.reshape(seq, batch, table.shape[1])
